# baseline (speedup 1.0000x reference)
import dataclasses
import functools

import jax
import jax.numpy as jnp
from jax import lax
from jax.experimental import pallas as pl
from jax.experimental.pallas import tpu as pltpu
from jax.experimental.pallas import tpu_sc as plsc

BS, M, N = 8, 1024, 4096
IN_DIM, SKIP_DIM, OUT_DIM = 256, 128, 256
NT = 512

NW = 32
PW = BS * N // NW
CP = 64
NCH = PW // CP


def _nn_body(xyz_ref, pt_ref, idx_ref, w_ref):
    b = pl.program_id(0)
    xyz = xyz_ref[0]
    pt = pt_ref[0]
    xx = jnp.sum(xyz * xyz, axis=1, keepdims=True)
    sq = pt * pt
    pp = (sq[0:1, :] + sq[1:2, :]) + sq[2:3, :]
    cross = jax.lax.dot_general(xyz, pt, (((1,), (0,)), ((), ())),
                                preferred_element_type=jnp.float32)
    d2 = (pp + xx) - 2.0 * cross
    iota0 = jax.lax.broadcasted_iota(jnp.int32, (M, NT), 0)
    dists, idxs = [], []
    for k in range(3):
        dmin = jnp.min(d2, axis=0, keepdims=True)
        idx = jnp.min(jnp.where(d2 == dmin, iota0, M), axis=0, keepdims=True)
        dists.append(dmin)
        idxs.append(idx)
        if k < 2:
            d2 = jnp.where(iota0 == idx, jnp.float32(3.4e38), d2)
    inv = [1.0 / (d + 1e-8) for d in dists]
    norm = inv[0] + inv[1] + inv[2]
    for k in range(3):
        idx_ref[0, k:k + 1, :] = idxs[k] + b * M
        w_ref[0, k:k + 1, :] = inv[k] / norm


def _three_nn_tc(xyz, parent_t):
    return pl.pallas_call(
        _nn_body,
        grid=(BS, N // NT),
        in_specs=[
            pl.BlockSpec((1, M, 3), lambda b, j: (b, 0, 0)),
            pl.BlockSpec((1, 3, NT), lambda b, j: (b, 0, j)),
        ],
        out_specs=[
            pl.BlockSpec((1, 3, NT), lambda b, j: (b, 0, j)),
            pl.BlockSpec((1, 3, NT), lambda b, j: (b, 0, j)),
        ],
        out_shape=[
            jax.ShapeDtypeStruct((BS, 3, N), jnp.int32),
            jax.ShapeDtypeStruct((BS, 3, N), jnp.float32),
        ],
    )(xyz, parent_t)


def _sc_interp(ft, idxg, w3):
    mesh = plsc.VectorSubcoreMesh(core_axis_name="c", subcore_axis_name="s")
    cp = pltpu.CompilerParams()
    if "needs_layout_passes" in pltpu.CompilerParams.__dataclass_fields__:
        cp = dataclasses.replace(cp, needs_layout_passes=False)

    @functools.partial(
        pl.kernel, mesh=mesh, compiler_params=cp,
        out_type=jax.ShapeDtypeStruct((BS * N, IN_DIM), jnp.float32),
        scratch_types=[
            pltpu.VMEM((CP,), jnp.int32),
            pltpu.VMEM((CP,), jnp.int32),
            pltpu.VMEM((CP,), jnp.int32),
            pltpu.VMEM((CP,), jnp.float32),
            pltpu.VMEM((CP,), jnp.float32),
            pltpu.VMEM((CP,), jnp.float32),
            pltpu.VMEM((CP, IN_DIM), jnp.float32),
            pltpu.VMEM((CP, IN_DIM), jnp.float32),
            pltpu.VMEM((CP, IN_DIM), jnp.float32),
            pltpu.VMEM((CP, IN_DIM), jnp.float32),
            pltpu.SemaphoreType.DMA,
        ],
    )
    def k(ft_hbm, idx_hbm, w_hbm, out_hbm, i0, i1, i2, w0v, w1v, w2v,
          g0, g1, g2, ov, sem):
        wid = lax.axis_index("s") * 2 + lax.axis_index("c")
        base = wid * PW
        b = base // N
        n0 = base % N

        @pl.loop(0, NCH)
        def _(c):
            ns = n0 + c * CP
            pltpu.sync_copy(idx_hbm.at[b, 0, pl.ds(ns, CP)], i0)
            pltpu.sync_copy(idx_hbm.at[b, 1, pl.ds(ns, CP)], i1)
            pltpu.sync_copy(idx_hbm.at[b, 2, pl.ds(ns, CP)], i2)
            pltpu.sync_copy(w_hbm.at[b, 0, pl.ds(ns, CP)], w0v)
            pltpu.sync_copy(w_hbm.at[b, 1, pl.ds(ns, CP)], w1v)
            pltpu.sync_copy(w_hbm.at[b, 2, pl.ds(ns, CP)], w2v)
            pltpu.async_copy(ft_hbm.at[i0], g0, sem).wait()
            pltpu.async_copy(ft_hbm.at[i1], g1, sem).wait()
            pltpu.async_copy(ft_hbm.at[i2], g2, sem).wait()

            @pl.loop(0, CP)
            def _(p):
                psp = jnp.full((16,), p, jnp.int32)
                w0 = plsc.load_gather(w0v, [psp])
                w1 = plsc.load_gather(w1v, [psp])
                w2 = plsc.load_gather(w2v, [psp])
                for s in range(IN_DIM // 16):
                    sl = pl.ds(s * 16, 16)
                    ov[p, sl] = ((g0[p, sl] * w0 + g1[p, sl] * w1)
                                 + g2[p, sl] * w2)

            pltpu.sync_copy(ov, out_hbm.at[pl.ds(base + c * CP, CP)])

    return k(ft, idxg, w3)


def _mlp_body(it_ref, st_ref, w1at_ref, w1bt_ref, b1_ref, w2_ref, b2_ref,
              o_ref):
    it = it_ref[0]
    st = st_ref[0]
    h = (jnp.dot(it, w1at_ref[...], preferred_element_type=jnp.float32)
         + jnp.dot(st, w1bt_ref[...], preferred_element_type=jnp.float32)
         + b1_ref[...])
    h = jnp.maximum(h, 0.0)
    out = jax.lax.dot_general(w2_ref[...], h, (((1,), (1,)), ((), ())),
                              preferred_element_type=jnp.float32) + b2_ref[...]
    o_ref[0] = jnp.maximum(out, 0.0)


def _mlp_tc(interp_t, skip_t, W1, b1, W2, b2):
    w1at = W1[:, :IN_DIM].T
    w1bt = W1[:, IN_DIM:].T
    b1r = b1.reshape(1, OUT_DIM)
    b2c = b2.reshape(OUT_DIM, 1)
    return pl.pallas_call(
        _mlp_body,
        grid=(BS, N // NT),
        in_specs=[
            pl.BlockSpec((1, NT, IN_DIM), lambda b, j: (b, j, 0)),
            pl.BlockSpec((1, NT, SKIP_DIM), lambda b, j: (b, j, 0)),
            pl.BlockSpec((IN_DIM, OUT_DIM), lambda b, j: (0, 0)),
            pl.BlockSpec((SKIP_DIM, OUT_DIM), lambda b, j: (0, 0)),
            pl.BlockSpec((1, OUT_DIM), lambda b, j: (0, 0)),
            pl.BlockSpec((OUT_DIM, OUT_DIM), lambda b, j: (0, 0)),
            pl.BlockSpec((OUT_DIM, 1), lambda b, j: (0, 0)),
        ],
        out_specs=pl.BlockSpec((1, OUT_DIM, NT), lambda b, j: (b, 0, j)),
        out_shape=jax.ShapeDtypeStruct((BS, OUT_DIM, N), jnp.float32),
    )(interp_t, skip_t, w1at, w1bt, b1r, W2, b2c)


@jax.jit
def kernel(xyz, parent_xyz, feats, skip_feats, W1, b1, W2, b2):
    parent_t = parent_xyz.transpose(0, 2, 1)
    ft = feats.transpose(0, 2, 1).reshape(BS * M, IN_DIM)
    skip_t = skip_feats.transpose(0, 2, 1)
    idxg, w3 = _three_nn_tc(xyz, parent_t)
    interp_rows = _sc_interp(ft, idxg, w3)
    interp_t = interp_rows.reshape(BS, N, IN_DIM)
    return _mlp_tc(interp_t, skip_t, W1, b1, W2, b2)


# speedup vs baseline: 1.3386x; 1.3386x over previous
import dataclasses
import functools

import jax
import jax.numpy as jnp
from jax import lax
from jax.experimental import pallas as pl
from jax.experimental.pallas import tpu as pltpu
from jax.experimental.pallas import tpu_sc as plsc

BS, M, N = 8, 1024, 4096
IN_DIM, SKIP_DIM, OUT_DIM = 256, 128, 256
NT = 512

NW = 32
CP = 64
GB = 2


def _make_nn_body(row_offset):
    def _nn_body(xyz_ref, pt_ref, idx_ref, w_ref):
        b = pl.program_id(0)
        xyz = xyz_ref[0]
        pt = pt_ref[0]
        xx = jnp.sum(xyz * xyz, axis=1, keepdims=True)
        sq = pt * pt
        pp = (sq[0:1, :] + sq[1:2, :]) + sq[2:3, :]
        cross = jax.lax.dot_general(xyz, pt, (((1,), (0,)), ((), ())),
                                    preferred_element_type=jnp.float32)
        d2 = (pp + xx) - 2.0 * cross
        iota0 = jax.lax.broadcasted_iota(jnp.int32, (M, NT), 0)
        dists, idxs = [], []
        for k in range(3):
            dmin = jnp.min(d2, axis=0, keepdims=True)
            idx = jnp.min(jnp.where(d2 == dmin, iota0, M), axis=0,
                          keepdims=True)
            dists.append(dmin)
            idxs.append(idx)
            if k < 2:
                d2 = jnp.where(iota0 == idx, jnp.float32(3.4e38), d2)
        inv = [1.0 / (d + 1e-8) for d in dists]
        norm = inv[0] + inv[1] + inv[2]
        for k in range(3):
            idx_ref[0, k:k + 1, :] = idxs[k] + (row_offset + b * M)
            w_ref[0, k:k + 1, :] = inv[k] / norm
    return _nn_body


def _three_nn_tc(xyz, parent_t, row_offset, gb):
    return pl.pallas_call(
        _make_nn_body(row_offset),
        grid=(gb, N // NT),
        in_specs=[
            pl.BlockSpec((1, M, 3), lambda b, j: (b, 0, 0)),
            pl.BlockSpec((1, 3, NT), lambda b, j: (b, 0, j)),
        ],
        out_specs=[
            pl.BlockSpec((1, 3, NT), lambda b, j: (b, 0, j)),
            pl.BlockSpec((1, 3, NT), lambda b, j: (b, 0, j)),
        ],
        out_shape=[
            jax.ShapeDtypeStruct((gb, 3, N), jnp.int32),
            jax.ShapeDtypeStruct((gb, 3, N), jnp.float32),
        ],
    )(xyz, parent_t)


def _sc_interp(ft, idxg, w3, gb):
    mesh = plsc.VectorSubcoreMesh(core_axis_name="c", subcore_axis_name="s")
    cp = pltpu.CompilerParams()
    if "needs_layout_passes" in pltpu.CompilerParams.__dataclass_fields__:
        cp = dataclasses.replace(cp, needs_layout_passes=False)
    pw = gb * N // NW
    nch = pw // CP

    @functools.partial(
        pl.kernel, mesh=mesh, compiler_params=cp,
        out_type=jax.ShapeDtypeStruct((gb * N, IN_DIM), jnp.float32),
        scratch_types=[
            pltpu.VMEM((CP,), jnp.int32),
            pltpu.VMEM((CP,), jnp.int32),
            pltpu.VMEM((CP,), jnp.int32),
            pltpu.VMEM((CP,), jnp.float32),
            pltpu.VMEM((CP,), jnp.float32),
            pltpu.VMEM((CP,), jnp.float32),
            pltpu.VMEM((CP, IN_DIM), jnp.float32),
            pltpu.VMEM((CP, IN_DIM), jnp.float32),
            pltpu.VMEM((CP, IN_DIM), jnp.float32),
            pltpu.VMEM((CP, IN_DIM), jnp.float32),
            pltpu.SemaphoreType.DMA,
        ],
    )
    def k(ft_hbm, idx_hbm, w_hbm, out_hbm, i0, i1, i2, w0v, w1v, w2v,
          g0, g1, g2, ov, sem):
        wid = lax.axis_index("s") * 2 + lax.axis_index("c")
        base = wid * pw
        b = base // N
        n0 = base % N

        @pl.loop(0, nch)
        def _(c):
            ns = n0 + c * CP
            pltpu.sync_copy(idx_hbm.at[b, 0, pl.ds(ns, CP)], i0)
            pltpu.sync_copy(idx_hbm.at[b, 1, pl.ds(ns, CP)], i1)
            pltpu.sync_copy(idx_hbm.at[b, 2, pl.ds(ns, CP)], i2)
            pltpu.sync_copy(w_hbm.at[b, 0, pl.ds(ns, CP)], w0v)
            pltpu.sync_copy(w_hbm.at[b, 1, pl.ds(ns, CP)], w1v)
            pltpu.sync_copy(w_hbm.at[b, 2, pl.ds(ns, CP)], w2v)
            pltpu.async_copy(ft_hbm.at[i0], g0, sem).wait()
            pltpu.async_copy(ft_hbm.at[i1], g1, sem).wait()
            pltpu.async_copy(ft_hbm.at[i2], g2, sem).wait()

            @pl.loop(0, CP)
            def _(p):
                psp = jnp.full((16,), p, jnp.int32)
                w0 = plsc.load_gather(w0v, [psp])
                w1 = plsc.load_gather(w1v, [psp])
                w2 = plsc.load_gather(w2v, [psp])
                for s in range(IN_DIM // 16):
                    sl = pl.ds(s * 16, 16)
                    ov[p, sl] = ((g0[p, sl] * w0 + g1[p, sl] * w1)
                                 + g2[p, sl] * w2)

            pltpu.sync_copy(ov, out_hbm.at[pl.ds(base + c * CP, CP)])

    return k(ft, idxg, w3)


def _mlp_body(it_ref, st_ref, w1at_ref, w1bt_ref, b1_ref, w2_ref, b2_ref,
              o_ref):
    it = it_ref[0]
    st = st_ref[0]
    h = (jnp.dot(it, w1at_ref[...], preferred_element_type=jnp.float32)
         + jnp.dot(st, w1bt_ref[...], preferred_element_type=jnp.float32)
         + b1_ref[...])
    h = jnp.maximum(h, 0.0)
    out = jax.lax.dot_general(w2_ref[...], h, (((1,), (1,)), ((), ())),
                              preferred_element_type=jnp.float32) + b2_ref[...]
    o_ref[0] = jnp.maximum(out, 0.0)


def _mlp_tc(interp_t, skip_t, W1, b1, W2, b2, gb):
    w1at = W1[:, :IN_DIM].T
    w1bt = W1[:, IN_DIM:].T
    b1r = b1.reshape(1, OUT_DIM)
    b2c = b2.reshape(OUT_DIM, 1)
    return pl.pallas_call(
        _mlp_body,
        grid=(gb, N // NT),
        in_specs=[
            pl.BlockSpec((1, NT, IN_DIM), lambda b, j: (b, j, 0)),
            pl.BlockSpec((1, NT, SKIP_DIM), lambda b, j: (b, j, 0)),
            pl.BlockSpec((IN_DIM, OUT_DIM), lambda b, j: (0, 0)),
            pl.BlockSpec((SKIP_DIM, OUT_DIM), lambda b, j: (0, 0)),
            pl.BlockSpec((1, OUT_DIM), lambda b, j: (0, 0)),
            pl.BlockSpec((OUT_DIM, OUT_DIM), lambda b, j: (0, 0)),
            pl.BlockSpec((OUT_DIM, 1), lambda b, j: (0, 0)),
        ],
        out_specs=pl.BlockSpec((1, OUT_DIM, NT), lambda b, j: (b, 0, j)),
        out_shape=jax.ShapeDtypeStruct((gb, OUT_DIM, N), jnp.float32),
    )(interp_t, skip_t, w1at, w1bt, b1r, W2, b2c)


@jax.jit
def kernel(xyz, parent_xyz, feats, skip_feats, W1, b1, W2, b2):
    parent_t = parent_xyz.transpose(0, 2, 1)
    ft = feats.transpose(0, 2, 1).reshape(BS * M, IN_DIM)
    skip_t = skip_feats.transpose(0, 2, 1)
    outs = []
    for g in range(BS // GB):
        s = slice(g * GB, (g + 1) * GB)
        idxg, w3 = _three_nn_tc(xyz[s], parent_t[s], g * GB * M, GB)
        interp_rows = _sc_interp(ft, idxg, w3, GB)
        interp_t = interp_rows.reshape(GB, N, IN_DIM)
        outs.append(_mlp_tc(interp_t, skip_t[s], W1, b1, W2, b2, GB))
    return jnp.concatenate(outs, axis=0)


# speedup vs baseline: 1.3916x; 1.0396x over previous
import dataclasses
import functools

import jax
import jax.numpy as jnp
from jax import lax
from jax.experimental import pallas as pl
from jax.experimental.pallas import tpu as pltpu
from jax.experimental.pallas import tpu_sc as plsc

BS, M, N = 8, 1024, 4096
IN_DIM, SKIP_DIM, OUT_DIM = 256, 128, 256
NT = 512

NW = 32
CP = 64
GB = 1


def _make_nn_body(row_offset):
    def _nn_body(xyz_ref, pt_ref, idx_ref, w_ref):
        b = pl.program_id(0)
        xyz = xyz_ref[0]
        pt = pt_ref[0]
        xx = jnp.sum(xyz * xyz, axis=1, keepdims=True)
        sq = pt * pt
        pp = (sq[0:1, :] + sq[1:2, :]) + sq[2:3, :]
        cross = jax.lax.dot_general(xyz, pt, (((1,), (0,)), ((), ())),
                                    preferred_element_type=jnp.float32)
        d2 = (pp + xx) - 2.0 * cross
        iota0 = jax.lax.broadcasted_iota(jnp.int32, (M, NT), 0)
        dists, idxs = [], []
        for k in range(3):
            dmin = jnp.min(d2, axis=0, keepdims=True)
            idx = jnp.min(jnp.where(d2 == dmin, iota0, M), axis=0,
                          keepdims=True)
            dists.append(dmin)
            idxs.append(idx)
            if k < 2:
                d2 = jnp.where(iota0 == idx, jnp.float32(3.4e38), d2)
        inv = [1.0 / (d + 1e-8) for d in dists]
        norm = inv[0] + inv[1] + inv[2]
        for k in range(3):
            idx_ref[0, k:k + 1, :] = idxs[k] + (row_offset + b * M)
            w_ref[0, k:k + 1, :] = inv[k] / norm
    return _nn_body


def _three_nn_tc(xyz, parent_t, row_offset, gb):
    return pl.pallas_call(
        _make_nn_body(row_offset),
        grid=(gb, N // NT),
        in_specs=[
            pl.BlockSpec((1, M, 3), lambda b, j: (b, 0, 0)),
            pl.BlockSpec((1, 3, NT), lambda b, j: (b, 0, j)),
        ],
        out_specs=[
            pl.BlockSpec((1, 3, NT), lambda b, j: (b, 0, j)),
            pl.BlockSpec((1, 3, NT), lambda b, j: (b, 0, j)),
        ],
        out_shape=[
            jax.ShapeDtypeStruct((gb, 3, N), jnp.int32),
            jax.ShapeDtypeStruct((gb, 3, N), jnp.float32),
        ],
    )(xyz, parent_t)


def _sc_interp(ft, idxg, w3, gb):
    mesh = plsc.VectorSubcoreMesh(core_axis_name="c", subcore_axis_name="s")
    cp = pltpu.CompilerParams()
    if "needs_layout_passes" in pltpu.CompilerParams.__dataclass_fields__:
        cp = dataclasses.replace(cp, needs_layout_passes=False)
    pw = gb * N // NW
    nch = pw // CP

    @functools.partial(
        pl.kernel, mesh=mesh, compiler_params=cp,
        out_type=jax.ShapeDtypeStruct((gb * N, IN_DIM), jnp.float32),
        scratch_types=[
            pltpu.VMEM((CP,), jnp.int32),
            pltpu.VMEM((CP,), jnp.int32),
            pltpu.VMEM((CP,), jnp.int32),
            pltpu.VMEM((CP,), jnp.float32),
            pltpu.VMEM((CP,), jnp.float32),
            pltpu.VMEM((CP,), jnp.float32),
            pltpu.VMEM((CP, IN_DIM), jnp.float32),
            pltpu.VMEM((CP, IN_DIM), jnp.float32),
            pltpu.VMEM((CP, IN_DIM), jnp.float32),
            pltpu.VMEM((CP, IN_DIM), jnp.float32),
            pltpu.SemaphoreType.DMA,
        ],
    )
    def k(ft_hbm, idx_hbm, w_hbm, out_hbm, i0, i1, i2, w0v, w1v, w2v,
          g0, g1, g2, ov, sem):
        wid = lax.axis_index("s") * 2 + lax.axis_index("c")
        base = wid * pw
        b = base // N
        n0 = base % N

        @pl.loop(0, nch)
        def _(c):
            ns = n0 + c * CP
            pltpu.sync_copy(idx_hbm.at[b, 0, pl.ds(ns, CP)], i0)
            pltpu.sync_copy(idx_hbm.at[b, 1, pl.ds(ns, CP)], i1)
            pltpu.sync_copy(idx_hbm.at[b, 2, pl.ds(ns, CP)], i2)
            pltpu.sync_copy(w_hbm.at[b, 0, pl.ds(ns, CP)], w0v)
            pltpu.sync_copy(w_hbm.at[b, 1, pl.ds(ns, CP)], w1v)
            pltpu.sync_copy(w_hbm.at[b, 2, pl.ds(ns, CP)], w2v)
            pltpu.async_copy(ft_hbm.at[i0], g0, sem).wait()
            pltpu.async_copy(ft_hbm.at[i1], g1, sem).wait()
            pltpu.async_copy(ft_hbm.at[i2], g2, sem).wait()

            @pl.loop(0, CP)
            def _(p):
                psp = jnp.full((16,), p, jnp.int32)
                w0 = plsc.load_gather(w0v, [psp])
                w1 = plsc.load_gather(w1v, [psp])
                w2 = plsc.load_gather(w2v, [psp])
                for s in range(IN_DIM // 16):
                    sl = pl.ds(s * 16, 16)
                    ov[p, sl] = ((g0[p, sl] * w0 + g1[p, sl] * w1)
                                 + g2[p, sl] * w2)

            pltpu.sync_copy(ov, out_hbm.at[pl.ds(base + c * CP, CP)])

    return k(ft, idxg, w3)


def _mlp_body(it_ref, st_ref, w1at_ref, w1bt_ref, b1_ref, w2_ref, b2_ref,
              o_ref):
    it = it_ref[0]
    st = st_ref[0]
    h = (jnp.dot(it, w1at_ref[...], preferred_element_type=jnp.float32)
         + jnp.dot(st, w1bt_ref[...], preferred_element_type=jnp.float32)
         + b1_ref[...])
    h = jnp.maximum(h, 0.0)
    out = jax.lax.dot_general(w2_ref[...], h, (((1,), (1,)), ((), ())),
                              preferred_element_type=jnp.float32) + b2_ref[...]
    o_ref[0] = jnp.maximum(out, 0.0)


def _mlp_tc(interp_t, skip_t, W1, b1, W2, b2, gb):
    w1at = W1[:, :IN_DIM].T
    w1bt = W1[:, IN_DIM:].T
    b1r = b1.reshape(1, OUT_DIM)
    b2c = b2.reshape(OUT_DIM, 1)
    return pl.pallas_call(
        _mlp_body,
        grid=(gb, N // NT),
        in_specs=[
            pl.BlockSpec((1, NT, IN_DIM), lambda b, j: (b, j, 0)),
            pl.BlockSpec((1, NT, SKIP_DIM), lambda b, j: (b, j, 0)),
            pl.BlockSpec((IN_DIM, OUT_DIM), lambda b, j: (0, 0)),
            pl.BlockSpec((SKIP_DIM, OUT_DIM), lambda b, j: (0, 0)),
            pl.BlockSpec((1, OUT_DIM), lambda b, j: (0, 0)),
            pl.BlockSpec((OUT_DIM, OUT_DIM), lambda b, j: (0, 0)),
            pl.BlockSpec((OUT_DIM, 1), lambda b, j: (0, 0)),
        ],
        out_specs=pl.BlockSpec((1, OUT_DIM, NT), lambda b, j: (b, 0, j)),
        out_shape=jax.ShapeDtypeStruct((gb, OUT_DIM, N), jnp.float32),
    )(interp_t, skip_t, w1at, w1bt, b1r, W2, b2c)


@jax.jit
def kernel(xyz, parent_xyz, feats, skip_feats, W1, b1, W2, b2):
    parent_t = parent_xyz.transpose(0, 2, 1)
    ft = feats.transpose(0, 2, 1).reshape(BS * M, IN_DIM)
    skip_t = skip_feats.transpose(0, 2, 1)
    outs = []
    for g in range(BS // GB):
        s = slice(g * GB, (g + 1) * GB)
        idxg, w3 = _three_nn_tc(xyz[s], parent_t[s], g * GB * M, GB)
        interp_rows = _sc_interp(ft, idxg, w3, GB)
        interp_t = interp_rows.reshape(GB, N, IN_DIM)
        outs.append(_mlp_tc(interp_t, skip_t[s], W1, b1, W2, b2, GB))
    return jnp.concatenate(outs, axis=0)
